# baseline (speedup 1.0000x reference)
import functools

import jax
import jax.numpy as jnp
from jax import lax
from jax.experimental import pallas as pl
from jax.experimental.pallas import tpu as pltpu
from jax.experimental.pallas import tpu_sc as plsc

_T = 2048
_V = 100000
_NW = 32
_BPW = _T // _NW
_L = 16

_V_BLK = 1024


@functools.partial(
    pl.kernel,
    out_type=jax.ShapeDtypeStruct((_T,), jnp.float32),
    mesh=plsc.VectorSubcoreMesh(core_axis_name="c", subcore_axis_name="s"),
    scratch_types=[
        pltpu.VMEM((_BPW,), jnp.int32),
        pltpu.VMEM((_BPW,), jnp.float32),
        pltpu.VMEM((_V,), jnp.float32),
        pltpu.VMEM((_BPW,), jnp.float32),
    ],
    compiler_params=pltpu.CompilerParams(needs_layout_passes=False),
)
def _sc_embed(idx_hbm, pos_hbm, table_hbm, out_hbm, idx_v, pos_v, table_v, out_v):
    wid = lax.axis_index("s") * 2 + lax.axis_index("c")
    base = wid * _BPW
    pltpu.sync_copy(table_hbm, table_v)
    pltpu.sync_copy(idx_hbm.at[pl.ds(base, _BPW)], idx_v)
    pltpu.sync_copy(pos_hbm.at[pl.ds(base, _BPW)], pos_v)
    for i in range(_BPW // _L):
        sl = pl.ds(i * _L, _L)
        vals = plsc.load_gather(table_v, [idx_v[sl]])
        out_v[sl] = vals + pos_v[sl]
    pltpu.sync_copy(out_v, out_hbm.at[pl.ds(base, _BPW)])


def _proj_body(x_ref, w_ref, b_ref, o_ref):
    o_ref[...] = x_ref[...] * w_ref[...] + b_ref[...]


def kernel(idx, tok_table, pos_emb, W, b):
    T = idx.shape[1]
    V = W.shape[0]
    idx_flat = idx.reshape(T).astype(jnp.int32)

    x = _sc_embed(idx_flat, pos_emb.reshape(T), tok_table.reshape(V))

    out = pl.pallas_call(
        _proj_body,
        grid=(pl.cdiv(V, _V_BLK),),
        in_specs=[
            pl.BlockSpec((T, 1), lambda j: (0, 0)),
            pl.BlockSpec((1, _V_BLK), lambda j: (0, j)),
            pl.BlockSpec((1, _V_BLK), lambda j: (0, j)),
        ],
        out_specs=pl.BlockSpec((T, _V_BLK), lambda j: (0, j)),
        out_shape=jax.ShapeDtypeStruct((T, V), jnp.float32),
        compiler_params=pltpu.CompilerParams(
            dimension_semantics=("arbitrary",),
        ),
    )(x.reshape(T, 1), W.reshape(1, V), b.reshape(1, V))
    return out.reshape(1, T, V)
